# Initial kernel scaffold; baseline (speedup 1.0000x reference)
import jax
import jax.numpy as jnp
from jax.experimental import pallas as pl


def kernel(logits, labels):
    raise NotImplementedError("write your pallas kernel here")


# speedup vs baseline: 1.0626x; 1.0626x over previous
import functools

import jax
import jax.numpy as jnp
from jax import lax
from jax.experimental import pallas as pl
from jax.experimental.pallas import tpu as pltpu
from jax.experimental.pallas import tpu_sc as plsc

_NB = 10
_N = 16777216
_NC = 2
_NS = 16
_NW = _NC * _NS
_L = 16
_PER_W = _N // _NW
_C = 16384
_NCHUNK = _PER_W // _C
_ACC = _NB * _L


def _sc_body(logits_hbm, labels_hbm, out_hbm,
             lbuf0, lbuf1, bbuf0, bbuf1,
             acc_cnt, acc_lab, acc_conf, sem0, sem1):
    i32 = jnp.int32
    wid = lax.axis_index("s") * i32(_NC) + lax.axis_index("c")
    base = wid * i32(_PER_W)

    zf = jnp.zeros((_L,), jnp.float32)
    for k in range(_NB):
        acc_cnt[pl.ds(k * _L, _L)] = zf
        acc_lab[pl.ds(k * _L, _L)] = zf
        acc_conf[pl.ds(k * _L, _L)] = zf

    def start(i, lbuf, bbuf, sem):
        off = base + i * i32(_C)
        pltpu.async_copy(logits_hbm.at[pl.ds(off, _C)], lbuf, sem)
        pltpu.async_copy(labels_hbm.at[pl.ds(off, _C)], bbuf, sem)

    def wait(lbuf, bbuf, sem):
        pltpu.make_async_copy(logits_hbm.at[pl.ds(0, _C)], lbuf, sem).wait()
        pltpu.make_async_copy(labels_hbm.at[pl.ds(0, _C)], bbuf, sem).wait()

    start(0, lbuf0, bbuf0, sem0)
    start(1, lbuf1, bbuf1, sem1)

    lane = lax.iota(jnp.int32, _L)
    ones = jnp.ones((_L,), jnp.float32)

    def consume(lbuf, bbuf):
        def inner(j, carry):
            base_e = j * i32(128)
            for u in range(8):
                off = base_e + i32(u * _L)
                x = lbuf[pl.ds(off, _L)]
                lab = bbuf[pl.ds(off, _L)]
                conf = 1.0 / (1.0 + jnp.exp(-x))
                bi = jnp.minimum((conf * 10.0).astype(jnp.int32), i32(9))
                addr = bi * i32(_L) + lane
                plsc.addupdate_scatter(acc_cnt, [addr], ones)
                plsc.addupdate_scatter(acc_lab, [addr],
                                       lab.astype(jnp.float32))
                plsc.addupdate_scatter(acc_conf, [addr], conf)
            return carry
        lax.fori_loop(i32(0), i32(_C // 128), inner, i32(0))

    def outer(t, carry):
        i0 = t * i32(2)
        wait(lbuf0, bbuf0, sem0)
        consume(lbuf0, bbuf0)

        @pl.when(i0 + i32(2) < i32(_NCHUNK))
        def _():
            start(i0 + i32(2), lbuf0, bbuf0, sem0)

        wait(lbuf1, bbuf1, sem1)
        consume(lbuf1, bbuf1)

        @pl.when(i0 + i32(3) < i32(_NCHUNK))
        def _():
            start(i0 + i32(3), lbuf1, bbuf1, sem1)

        return carry

    lax.fori_loop(i32(0), i32(_NCHUNK // 2), outer, i32(0))

    obase = wid * i32(3 * _ACC)
    pltpu.sync_copy(acc_cnt, out_hbm.at[pl.ds(obase, _ACC)])
    pltpu.sync_copy(acc_lab, out_hbm.at[pl.ds(obase + i32(_ACC), _ACC)])
    pltpu.sync_copy(acc_conf, out_hbm.at[pl.ds(obase + i32(2 * _ACC), _ACC)])


@jax.jit
def _sc_hist(logits, labels):
    mesh = plsc.VectorSubcoreMesh(core_axis_name="c", subcore_axis_name="s",
                                  num_cores=_NC, num_subcores=_NS)
    f = pl.kernel(
        _sc_body,
        out_type=jax.ShapeDtypeStruct((_NW * 3 * _ACC,), jnp.float32),
        mesh=mesh,
        scratch_types=[
            pltpu.VMEM((_C,), jnp.float32),
            pltpu.VMEM((_C,), jnp.float32),
            pltpu.VMEM((_C,), jnp.int32),
            pltpu.VMEM((_C,), jnp.int32),
            pltpu.VMEM((_ACC,), jnp.float32),
            pltpu.VMEM((_ACC,), jnp.float32),
            pltpu.VMEM((_ACC,), jnp.float32),
            pltpu.SemaphoreType.DMA,
            pltpu.SemaphoreType.DMA,
        ],
        compiler_params=pltpu.CompilerParams(needs_layout_passes=False),
    )
    return f(logits, labels)


def _combine_body(p_ref, pc_ref, e_ref, m_ref):
    x = p_ref[...]
    s = jnp.sum(x, axis=(0, 3))
    cnt = s[0]
    lab = s[1]
    cf = s[2]
    nonempty = cnt > 0.0
    denom = jnp.maximum(cnt, 1.0)
    pos = jnp.where(nonempty, lab / denom, 0.0)
    cfm = jnp.where(nonempty, cf / denom, 0.0)
    ece_i = jnp.abs(pos - cfm)
    pc_ref[...] = jnp.stack([pos, cfm])
    e_ref[...] = jnp.sum(ece_i).reshape(1, 1)
    m_ref[...] = jnp.max(ece_i).reshape(1, 1)


def kernel(logits, labels):
    if labels.dtype != jnp.int32:
        labels = labels.astype(jnp.int32)
    partials = _sc_hist(logits, labels)
    p4 = partials.reshape(_NW, 3, _NB, _L)
    pc, e, m = pl.pallas_call(
        _combine_body,
        out_shape=[
            jax.ShapeDtypeStruct((2, _NB), jnp.float32),
            jax.ShapeDtypeStruct((1, 1), jnp.float32),
            jax.ShapeDtypeStruct((1, 1), jnp.float32),
        ],
    )(p4)
    return (pc[0], pc[1], e[0, 0], m[0, 0])


# speedup vs baseline: 2.0253x; 1.9060x over previous
import numpy as np

import jax
import jax.numpy as jnp
from jax import lax
from jax.experimental import pallas as pl
from jax.experimental.pallas import tpu as pltpu
from jax.experimental.pallas import tpu_sc as plsc

_NB = 10
_N = 16777216
_NC = 2
_NS = 16
_NW = _NC * _NS
_L = 16
_PER_W = _N // _NW
_C = 16384
_NCHUNK = _PER_W // _C
_ACC = _NB * _L

_TBL_N = 2048
_TBL_LO = -16.0
_TBL_SCALE = 64.0
_xs = _TBL_LO + np.arange(_TBL_N + 1, dtype=np.float64) / _TBL_SCALE
_sig = 1.0 / (1.0 + np.exp(-_xs))
_TBL_BASE = np.asarray(_sig[:-1], dtype=np.float32)
_TBL_SLOPE = np.asarray(_sig[1:] - _sig[:-1], dtype=np.float32)


def _sc_body(logits_hbm, labels_hbm, tb_hbm, ts_hbm, out_hbm,
             lbuf0, lbuf1, bbuf0, bbuf1, tb, ts,
             acc_cnt, acc_lab, acc_conf, sem0, sem1):
    i32 = jnp.int32
    wid = lax.axis_index("s") * i32(_NC) + lax.axis_index("c")
    base = wid * i32(_PER_W)

    pltpu.sync_copy(tb_hbm, tb)
    pltpu.sync_copy(ts_hbm, ts)

    zf = jnp.zeros((_L,), jnp.float32)
    for k in range(_NB):
        acc_cnt[pl.ds(k * _L, _L)] = zf
        acc_lab[pl.ds(k * _L, _L)] = zf
        acc_conf[pl.ds(k * _L, _L)] = zf

    def start(i, lbuf, bbuf, sem):
        off = base + i * i32(_C)
        pltpu.async_copy(logits_hbm.at[pl.ds(off, _C)], lbuf, sem)
        pltpu.async_copy(labels_hbm.at[pl.ds(off, _C)], bbuf, sem)

    def wait(lbuf, bbuf, sem):
        pltpu.make_async_copy(logits_hbm.at[pl.ds(0, _C)], lbuf, sem).wait()
        pltpu.make_async_copy(labels_hbm.at[pl.ds(0, _C)], bbuf, sem).wait()

    start(0, lbuf0, bbuf0, sem0)
    start(1, lbuf1, bbuf1, sem1)

    lane = lax.iota(jnp.int32, _L)
    ones = jnp.ones((_L,), jnp.float32)

    _U = 8

    def consume(lbuf, bbuf):
        def inner(j, carry):
            base_e = j * i32(_U * _L)
            offs = [base_e + i32(u * _L) for u in range(_U)]
            xs = [lbuf[pl.ds(o, _L)] for o in offs]
            labs = [bbuf[pl.ds(o, _L)] for o in offs]
            tts = [x * _TBL_SCALE + (-_TBL_LO * _TBL_SCALE) for x in xs]
            tts = [jnp.minimum(jnp.maximum(t, 0.0), _TBL_N - 0.004)
                   for t in tts]
            iis = [t.astype(jnp.int32) for t in tts]
            fracs = [t - i.astype(jnp.float32) for t, i in zip(tts, iis)]
            bas = [plsc.load_gather(tb, [i]) for i in iis]
            sls = [plsc.load_gather(ts, [i]) for i in iis]
            confs = [b + f * s for b, f, s in zip(bas, fracs, sls)]
            bis = [(c * 10.0).astype(jnp.int32) for c in confs]
            addrs = [b * i32(_L) + lane for b in bis]
            labfs = [l.astype(jnp.float32) for l in labs]
            for u in range(_U):
                plsc.addupdate_scatter(acc_cnt, [addrs[u]], ones)
                plsc.addupdate_scatter(acc_lab, [addrs[u]], labfs[u])
                plsc.addupdate_scatter(acc_conf, [addrs[u]], confs[u])
            return carry
        lax.fori_loop(i32(0), i32(_C // (_U * _L)), inner, i32(0))

    def outer(t, carry):
        i0 = t * i32(2)
        wait(lbuf0, bbuf0, sem0)
        consume(lbuf0, bbuf0)

        @pl.when(i0 + i32(2) < i32(_NCHUNK))
        def _():
            start(i0 + i32(2), lbuf0, bbuf0, sem0)

        wait(lbuf1, bbuf1, sem1)
        consume(lbuf1, bbuf1)

        @pl.when(i0 + i32(3) < i32(_NCHUNK))
        def _():
            start(i0 + i32(3), lbuf1, bbuf1, sem1)

        return carry

    lax.fori_loop(i32(0), i32(_NCHUNK // 2), outer, i32(0))

    obase = wid * i32(3 * _ACC)
    pltpu.sync_copy(acc_cnt, out_hbm.at[pl.ds(obase, _ACC)])
    pltpu.sync_copy(acc_lab, out_hbm.at[pl.ds(obase + i32(_ACC), _ACC)])
    pltpu.sync_copy(acc_conf, out_hbm.at[pl.ds(obase + i32(2 * _ACC), _ACC)])


@jax.jit
def _sc_hist(logits, labels, tbl_base, tbl_slope):
    mesh = plsc.VectorSubcoreMesh(core_axis_name="c", subcore_axis_name="s",
                                  num_cores=_NC, num_subcores=_NS)
    f = pl.kernel(
        _sc_body,
        out_type=jax.ShapeDtypeStruct((_NW * 3 * _ACC,), jnp.float32),
        mesh=mesh,
        scratch_types=[
            pltpu.VMEM((_C,), jnp.float32),
            pltpu.VMEM((_C,), jnp.float32),
            pltpu.VMEM((_C,), jnp.int32),
            pltpu.VMEM((_C,), jnp.int32),
            pltpu.VMEM((_TBL_N,), jnp.float32),
            pltpu.VMEM((_TBL_N,), jnp.float32),
            pltpu.VMEM((_ACC,), jnp.float32),
            pltpu.VMEM((_ACC,), jnp.float32),
            pltpu.VMEM((_ACC,), jnp.float32),
            pltpu.SemaphoreType.DMA,
            pltpu.SemaphoreType.DMA,
        ],
        compiler_params=pltpu.CompilerParams(needs_layout_passes=False),
    )
    return f(logits, labels, tbl_base, tbl_slope)


def _combine_body(p_ref, pc_ref, e_ref, m_ref):
    x = p_ref[...]
    s = jnp.sum(x, axis=(0, 3))
    cnt = s[0]
    lab = s[1]
    cf = s[2]
    nonempty = cnt > 0.0
    denom = jnp.maximum(cnt, 1.0)
    pos = jnp.where(nonempty, lab / denom, 0.0)
    cfm = jnp.where(nonempty, cf / denom, 0.0)
    ece_i = jnp.abs(pos - cfm)
    pc_ref[...] = jnp.stack([pos, cfm])
    e_ref[...] = jnp.sum(ece_i).reshape(1, 1)
    m_ref[...] = jnp.max(ece_i).reshape(1, 1)


def kernel(logits, labels):
    if labels.dtype != jnp.int32:
        labels = labels.astype(jnp.int32)
    partials = _sc_hist(logits, labels,
                        jnp.asarray(_TBL_BASE), jnp.asarray(_TBL_SLOPE))
    p4 = partials.reshape(_NW, 3, _NB, _L)
    pc, e, m = pl.pallas_call(
        _combine_body,
        out_shape=[
            jax.ShapeDtypeStruct((2, _NB), jnp.float32),
            jax.ShapeDtypeStruct((1, 1), jnp.float32),
            jax.ShapeDtypeStruct((1, 1), jnp.float32),
        ],
    )(p4)
    return (pc[0], pc[1], e[0, 0], m[0, 0])
